# Initial kernel scaffold; baseline (speedup 1.0000x reference)
import jax
import jax.numpy as jnp
from jax.experimental import pallas as pl


def kernel(x, l0_bn1_gamma, l0_bn1_beta, l0_bn1_mean, l0_bn1_var, l0_conv1_w, l0_bn2_gamma, l0_bn2_beta, l0_bn2_mean, l0_bn2_var, l0_conv2_w, l0_short_w, l1_bn1_gamma, l1_bn1_beta, l1_bn1_mean, l1_bn1_var, l1_conv1_w, l1_bn2_gamma, l1_bn2_beta, l1_bn2_mean, l1_bn2_var, l1_conv2_w, l2_bn1_gamma, l2_bn1_beta, l2_bn1_mean, l2_bn1_var, l2_conv1_w, l2_bn2_gamma, l2_bn2_beta, l2_bn2_mean, l2_bn2_var, l2_conv2_w, l3_bn1_gamma, l3_bn1_beta, l3_bn1_mean, l3_bn1_var, l3_conv1_w, l3_bn2_gamma, l3_bn2_beta, l3_bn2_mean, l3_bn2_var, l3_conv2_w):
    raise NotImplementedError("write your pallas kernel here")


# speedup vs baseline: 1.0177x; 1.0177x over previous
import jax
import jax.numpy as jnp
from jax.experimental import pallas as pl
from jax.experimental.pallas import tpu as pltpu

_EPS = 1e-5
_B = 4

_BF = jnp.bfloat16
_F32 = jnp.float32


def _conv3x3_acc(pad_ref, w_ref, acc):
    b, hp, wp, cin = pad_ref.shape
    ho, wo = hp - 2, wp - 2
    m = b * ho * wo
    for dy in range(3):
        for dx in range(3):
            patch = pad_ref[:, dy:dy + ho, dx:dx + wo, :].reshape(m, cin)
            d = jnp.dot(patch, w_ref[dy, dx], preferred_element_type=_F32)
            acc = d if acc is None else d + acc
    return acc


def _net_kernel(x2_ref,
                s10_ref, h10_ref, wA_ref, wB_ref,
                s20_ref, h20_ref, w20_ref, sw_ref,
                s11_ref, h11_ref, w11_ref, s21_ref, h21_ref, w21_ref,
                s12_ref, h12_ref, w12_ref, s22_ref, h22_ref, w22_ref,
                s13_ref, h13_ref, w13_ref, s23_ref, h23_ref, w23_ref,
                o_ref,
                pe_ref, po_ref, acts_ref, padA_ref, padB_ref, xbuf_ref):
    b, h, wo, c2 = x2_ref.shape
    ho = h // 2
    cin = c2 // 2
    cout = o_ref.shape[-1]
    m = b * ho * wo

    act = jnp.maximum(x2_ref[...] * s10_ref[...] + h10_ref[...], 0.0)
    act4 = act.reshape(b, ho, 2, wo, c2)
    even = act4[:, :, 0]
    odd = act4[:, :, 1]

    acts_ref[...] = even[..., :cin].astype(_BF)

    pe_ref[...] = jnp.zeros_like(pe_ref)
    pe_ref[:, :, 1:wo + 1, :] = even.astype(_BF)
    po_ref[...] = jnp.zeros_like(po_ref)
    po_ref[:, 1:ho + 1, 1:wo + 1, :] = odd.astype(_BF)

    acc = None
    for dy, (buf, r0) in enumerate(((po_ref, 0), (pe_ref, 0), (po_ref, 1))):
        left = buf[:, r0:r0 + ho, 0:wo, :].reshape(m, c2)
        right = buf[:, r0:r0 + ho, 1:wo + 1, :].reshape(m, c2)
        dl = jnp.dot(left, wA_ref[dy], preferred_element_type=_F32)
        acc = dl if acc is None else dl + acc
        acc = jnp.dot(right, wB_ref[dy], preferred_element_type=_F32) + acc

    a2 = jnp.maximum(acc.reshape(b, ho, wo, cout) * s20_ref[...] + h20_ref[...], 0.0)
    padA_ref[...] = jnp.zeros_like(padA_ref)
    padA_ref[:, 1:ho + 1, 1:wo + 1, :] = a2.astype(_BF)
    acc2 = jnp.dot(acts_ref[...].reshape(m, cin), sw_ref[...],
                   preferred_element_type=_F32)
    acc2 = _conv3x3_acc(padA_ref, w20_ref, acc2)
    xbuf_ref[...] = acc2.reshape(b, ho, wo, cout)

    layers = ((s11_ref, h11_ref, w11_ref, s21_ref, h21_ref, w21_ref),
              (s12_ref, h12_ref, w12_ref, s22_ref, h22_ref, w22_ref),
              (s13_ref, h13_ref, w13_ref, s23_ref, h23_ref, w23_ref))
    for li, (s1, h1, w1, s2, h2, w2) in enumerate(layers):
        p_in, p_out = (padB_ref, padA_ref) if li % 2 == 0 else (padA_ref, padB_ref)
        a1 = jnp.maximum(xbuf_ref[...] * s1[...] + h1[...], 0.0)
        p_in[...] = jnp.zeros_like(p_in)
        p_in[:, 1:ho + 1, 1:wo + 1, :] = a1.astype(_BF)
        u = _conv3x3_acc(p_in, w1, None)
        a2 = jnp.maximum(u.reshape(b, ho, wo, cout) * s2[...] + h2[...], 0.0)
        p_out[...] = jnp.zeros_like(p_out)
        p_out[:, 1:ho + 1, 1:wo + 1, :] = a2.astype(_BF)
        v = _conv3x3_acc(p_out, w2, None)
        out = v.reshape(b, ho, wo, cout) + xbuf_ref[...]
        if li == 2:
            o_ref[...] = out.astype(o_ref.dtype)
        else:
            xbuf_ref[...] = out


def _fold_bn(gamma, beta, mean, var):
    scale = gamma / jnp.sqrt(var + _EPS)
    shift = beta - mean * scale
    return scale, shift


def kernel(x, l0_bn1_gamma, l0_bn1_beta, l0_bn1_mean, l0_bn1_var, l0_conv1_w, l0_bn2_gamma, l0_bn2_beta, l0_bn2_mean, l0_bn2_var, l0_conv2_w, l0_short_w, l1_bn1_gamma, l1_bn1_beta, l1_bn1_mean, l1_bn1_var, l1_conv1_w, l1_bn2_gamma, l1_bn2_beta, l1_bn2_mean, l1_bn2_var, l1_conv2_w, l2_bn1_gamma, l2_bn1_beta, l2_bn1_mean, l2_bn1_var, l2_conv1_w, l2_bn2_gamma, l2_bn2_beta, l2_bn2_mean, l2_bn2_var, l2_conv2_w, l3_bn1_gamma, l3_bn1_beta, l3_bn1_mean, l3_bn1_var, l3_conv1_w, l3_bn2_gamma, l3_bn2_beta, l3_bn2_mean, l3_bn2_var, l3_conv2_w):
    n, cin, h, wsp = x.shape
    cout = l0_conv1_w.shape[-1]
    ho, wo = h // 2, wsp // 2
    c2 = 2 * cin
    b = _B

    x2 = jnp.transpose(x, (0, 2, 3, 1)).reshape(n, h, wo, c2)

    s10, h10 = _fold_bn(l0_bn1_gamma, l0_bn1_beta, l0_bn1_mean, l0_bn1_var)
    s10 = jnp.concatenate([s10, s10]).reshape(1, 1, 1, c2)
    h10 = jnp.concatenate([h10, h10]).reshape(1, 1, 1, c2)
    wA = jnp.concatenate([jnp.zeros_like(l0_conv1_w[:, 0]), l0_conv1_w[:, 0]],
                         axis=1).astype(_BF)
    wB = jnp.concatenate([l0_conv1_w[:, 1], l0_conv1_w[:, 2]], axis=1).astype(_BF)

    def vec(s):
        return s.reshape(1, 1, 1, -1)

    s20, h20 = _fold_bn(l0_bn2_gamma, l0_bn2_beta, l0_bn2_mean, l0_bn2_var)
    s11, h11 = _fold_bn(l1_bn1_gamma, l1_bn1_beta, l1_bn1_mean, l1_bn1_var)
    s21, h21 = _fold_bn(l1_bn2_gamma, l1_bn2_beta, l1_bn2_mean, l1_bn2_var)
    s12, h12 = _fold_bn(l2_bn1_gamma, l2_bn1_beta, l2_bn1_mean, l2_bn1_var)
    s22, h22 = _fold_bn(l2_bn2_gamma, l2_bn2_beta, l2_bn2_mean, l2_bn2_var)
    s13, h13 = _fold_bn(l3_bn1_gamma, l3_bn1_beta, l3_bn1_mean, l3_bn1_var)
    s23, h23 = _fold_bn(l3_bn2_gamma, l3_bn2_beta, l3_bn2_mean, l3_bn2_var)

    args = [
        x2,
        s10, h10, wA, wB,
        vec(s20), vec(h20), l0_conv2_w.astype(_BF), l0_short_w.astype(_BF),
        vec(s11), vec(h11), l1_conv1_w.astype(_BF),
        vec(s21), vec(h21), l1_conv2_w.astype(_BF),
        vec(s12), vec(h12), l2_conv1_w.astype(_BF),
        vec(s22), vec(h22), l2_conv2_w.astype(_BF),
        vec(s13), vec(h13), l3_conv1_w.astype(_BF),
        vec(s23), vec(h23), l3_conv2_w.astype(_BF),
    ]

    def const4(shape):
        nd = len(shape)
        return pl.BlockSpec(shape, lambda bi: (0,) * nd)

    in_specs = [pl.BlockSpec((b, h, wo, c2), lambda bi: (bi, 0, 0, 0))]
    in_specs += [const4(a.shape) for a in args[1:]]

    out = pl.pallas_call(
        _net_kernel,
        out_shape=jax.ShapeDtypeStruct((n, ho, wo, cout), x.dtype),
        grid=(n // b,),
        in_specs=in_specs,
        out_specs=pl.BlockSpec((b, ho, wo, cout), lambda bi: (bi, 0, 0, 0)),
        scratch_shapes=[
            pltpu.VMEM((b, ho, wo + 1, c2), _BF),
            pltpu.VMEM((b, ho + 1, wo + 1, c2), _BF),
            pltpu.VMEM((b, ho, wo, cin), _BF),
            pltpu.VMEM((b, ho + 2, wo + 2, cout), _BF),
            pltpu.VMEM((b, ho + 2, wo + 2, cout), _BF),
            pltpu.VMEM((b, ho, wo, cout), _F32),
        ],
        compiler_params=pltpu.CompilerParams(
            dimension_semantics=("parallel",),
            vmem_limit_bytes=64 * 1024 * 1024,
        ),
    )(*args)

    return jnp.transpose(out, (0, 3, 1, 2))


# speedup vs baseline: 1.3602x; 1.3365x over previous
import jax
import jax.numpy as jnp
from jax.experimental import pallas as pl
from jax.experimental.pallas import tpu as pltpu

_EPS = 1e-5
_B = 4

_BF = jnp.bfloat16
_F32 = jnp.float32


def _im2col_dot(act, imcol_ref, wcat_ref):
    b, ho, wo, c = act.shape
    m = b * ho * wo
    zrow = jnp.zeros((b, 1, wo, c), _BF)
    zcol = jnp.zeros((b, ho, 1, c), _BF)
    for dy in range(3):
        for dx in range(3):
            blk = (dy * 3 + dx) * c
            if dy == 0:
                imcol_ref[:, 0:1, :, blk:blk + c] = zrow
            elif dy == 2:
                imcol_ref[:, ho - 1:ho, :, blk:blk + c] = zrow
            if dx == 0:
                imcol_ref[:, :, 0:1, blk:blk + c] = zcol
            elif dx == 2:
                imcol_ref[:, :, wo - 1:wo, blk:blk + c] = zcol
            h0, h1 = max(0, 1 - dy), min(ho, ho + 1 - dy)
            w0, w1 = max(0, 1 - dx), min(wo, wo + 1 - dx)
            imcol_ref[:, h0:h1, w0:w1, blk:blk + c] = (
                act[:, h0 + dy - 1:h1 + dy - 1, w0 + dx - 1:w1 + dx - 1, :])
    return jnp.dot(imcol_ref[...].reshape(m, 9 * c), wcat_ref[...],
                   preferred_element_type=_F32)


def _net_kernel(x2_ref,
                s10_ref, h10_ref, wc0_ref,
                s20_ref, h20_ref, w20_ref, sw_ref,
                s11_ref, h11_ref, w11_ref, s21_ref, h21_ref, w21_ref,
                s12_ref, h12_ref, w12_ref, s22_ref, h22_ref, w22_ref,
                s13_ref, h13_ref, w13_ref, s23_ref, h23_ref, w23_ref,
                o_ref,
                imA_ref, imB_ref, acts_ref, xbuf_ref):
    b, h, wo, c2 = x2_ref.shape
    ho = h // 2
    cin = c2 // 2
    cout = o_ref.shape[-1]
    m = b * ho * wo

    act = jnp.maximum(x2_ref[...] * s10_ref[...] + h10_ref[...], 0.0)
    act4 = act.reshape(b, ho, 2, wo, c2)
    even = act4[:, :, 0].astype(_BF)
    odd = act4[:, :, 1].astype(_BF)

    acts_ref[...] = even[..., :cin]

    zrow = jnp.zeros((b, 1, wo, c2), _BF)
    zcol = jnp.zeros((b, ho, 1, c2), _BF)
    for dy in range(3):
        for side in range(2):
            blk = (dy * 2 + side) * c2
            w0, w1 = (1, wo) if side == 0 else (0, wo)
            if dy == 0:
                imA_ref[:, 0:1, :, blk:blk + c2] = zrow
                src = odd[:, 0:ho - 1]
                h0, h1 = 1, ho
            else:
                src = even if dy == 1 else odd
                h0, h1 = 0, ho
            if side == 0:
                imA_ref[:, :, 0:1, blk:blk + c2] = zcol
            imA_ref[:, h0:h1, w0:w1, blk:blk + c2] = src[:, :, 0:w1 - w0, :]
    k0 = 6 * c2
    u = jnp.dot(imA_ref[:, :, :, 0:k0].reshape(m, k0), wc0_ref[...],
                preferred_element_type=_F32)

    a2 = jnp.maximum(u.reshape(b, ho, wo, cout) * s20_ref[...] + h20_ref[...],
                     0.0).astype(_BF)
    acc2 = _im2col_dot(a2, imB_ref, w20_ref)
    acc2 = jnp.dot(acts_ref[...].reshape(m, cin), sw_ref[...],
                   preferred_element_type=_F32) + acc2
    xbuf_ref[...] = acc2.reshape(b, ho, wo, cout)

    layers = ((s11_ref, h11_ref, w11_ref, s21_ref, h21_ref, w21_ref),
              (s12_ref, h12_ref, w12_ref, s22_ref, h22_ref, w22_ref),
              (s13_ref, h13_ref, w13_ref, s23_ref, h23_ref, w23_ref))
    for li, (s1, h1, w1, s2, h2, w2) in enumerate(layers):
        p1, p2 = (imA_ref, imB_ref) if li % 2 == 0 else (imB_ref, imA_ref)
        a1 = jnp.maximum(xbuf_ref[...] * s1[...] + h1[...], 0.0).astype(_BF)
        uu = _im2col_dot(a1, p1, w1)
        a2 = jnp.maximum(uu.reshape(b, ho, wo, cout) * s2[...] + h2[...],
                         0.0).astype(_BF)
        vv = _im2col_dot(a2, p2, w2)
        out = vv.reshape(b, ho, wo, cout) + xbuf_ref[...]
        if li == 2:
            o_ref[...] = out.astype(o_ref.dtype)
        else:
            xbuf_ref[...] = out


def _fold_bn(gamma, beta, mean, var):
    scale = gamma / jnp.sqrt(var + _EPS)
    shift = beta - mean * scale
    return scale, shift


def kernel(x, l0_bn1_gamma, l0_bn1_beta, l0_bn1_mean, l0_bn1_var, l0_conv1_w, l0_bn2_gamma, l0_bn2_beta, l0_bn2_mean, l0_bn2_var, l0_conv2_w, l0_short_w, l1_bn1_gamma, l1_bn1_beta, l1_bn1_mean, l1_bn1_var, l1_conv1_w, l1_bn2_gamma, l1_bn2_beta, l1_bn2_mean, l1_bn2_var, l1_conv2_w, l2_bn1_gamma, l2_bn1_beta, l2_bn1_mean, l2_bn1_var, l2_conv1_w, l2_bn2_gamma, l2_bn2_beta, l2_bn2_mean, l2_bn2_var, l2_conv2_w, l3_bn1_gamma, l3_bn1_beta, l3_bn1_mean, l3_bn1_var, l3_conv1_w, l3_bn2_gamma, l3_bn2_beta, l3_bn2_mean, l3_bn2_var, l3_conv2_w):
    n, cin, h, wsp = x.shape
    cout = l0_conv1_w.shape[-1]
    ho, wo = h // 2, wsp // 2
    c2 = 2 * cin
    b = _B

    x2 = jnp.transpose(x, (0, 2, 3, 1)).reshape(n, h, wo, c2)

    s10, h10 = _fold_bn(l0_bn1_gamma, l0_bn1_beta, l0_bn1_mean, l0_bn1_var)
    s10 = jnp.concatenate([s10, s10]).reshape(1, 1, 1, c2)
    h10 = jnp.concatenate([h10, h10]).reshape(1, 1, 1, c2)
    wc0 = jnp.stack(
        [jnp.concatenate([jnp.zeros_like(l0_conv1_w[:, 0]), l0_conv1_w[:, 0]],
                         axis=1),
         jnp.concatenate([l0_conv1_w[:, 1], l0_conv1_w[:, 2]], axis=1)],
        axis=1).reshape(6 * c2, cout).astype(_BF)

    def vec(s):
        return s.reshape(1, 1, 1, -1)

    def wcat(w):
        return w.reshape(-1, w.shape[-1]).astype(_BF)

    s20, h20 = _fold_bn(l0_bn2_gamma, l0_bn2_beta, l0_bn2_mean, l0_bn2_var)
    s11, h11 = _fold_bn(l1_bn1_gamma, l1_bn1_beta, l1_bn1_mean, l1_bn1_var)
    s21, h21 = _fold_bn(l1_bn2_gamma, l1_bn2_beta, l1_bn2_mean, l1_bn2_var)
    s12, h12 = _fold_bn(l2_bn1_gamma, l2_bn1_beta, l2_bn1_mean, l2_bn1_var)
    s22, h22 = _fold_bn(l2_bn2_gamma, l2_bn2_beta, l2_bn2_mean, l2_bn2_var)
    s13, h13 = _fold_bn(l3_bn1_gamma, l3_bn1_beta, l3_bn1_mean, l3_bn1_var)
    s23, h23 = _fold_bn(l3_bn2_gamma, l3_bn2_beta, l3_bn2_mean, l3_bn2_var)

    args = [
        x2,
        s10, h10, wc0,
        vec(s20), vec(h20), wcat(l0_conv2_w), l0_short_w.astype(_BF),
        vec(s11), vec(h11), wcat(l1_conv1_w),
        vec(s21), vec(h21), wcat(l1_conv2_w),
        vec(s12), vec(h12), wcat(l2_conv1_w),
        vec(s22), vec(h22), wcat(l2_conv2_w),
        vec(s13), vec(h13), wcat(l3_conv1_w),
        vec(s23), vec(h23), wcat(l3_conv2_w),
    ]

    def const(shape):
        nd = len(shape)
        return pl.BlockSpec(shape, lambda bi: (0,) * nd)

    in_specs = [pl.BlockSpec((b, h, wo, c2), lambda bi: (bi, 0, 0, 0))]
    in_specs += [const(a.shape) for a in args[1:]]

    out = pl.pallas_call(
        _net_kernel,
        out_shape=jax.ShapeDtypeStruct((n, ho, wo, cout), x.dtype),
        grid=(n // b,),
        in_specs=in_specs,
        out_specs=pl.BlockSpec((b, ho, wo, cout), lambda bi: (bi, 0, 0, 0)),
        scratch_shapes=[
            pltpu.VMEM((b, ho, wo, 9 * cout), _BF),
            pltpu.VMEM((b, ho, wo, 9 * cout), _BF),
            pltpu.VMEM((b, ho, wo, cin), _BF),
            pltpu.VMEM((b, ho, wo, cout), _F32),
        ],
        compiler_params=pltpu.CompilerParams(
            dimension_semantics=("parallel",),
            vmem_limit_bytes=64 * 1024 * 1024,
        ),
    )(*args)

    return jnp.transpose(out, (0, 3, 1, 2))


# speedup vs baseline: 1.5011x; 1.1036x over previous
import jax
import jax.numpy as jnp
from jax.experimental import pallas as pl
from jax.experimental.pallas import tpu as pltpu

_EPS = 1e-5
_B = 4

_BF = jnp.bfloat16
_F32 = jnp.float32


def _im2col_dot(act, imcol_ref, wcat_ref):
    b, ho, wo, c = act.shape
    m = b * ho * wo
    zrow = jnp.zeros((b, 1, wo, c), _BF)
    zcol = jnp.zeros((b, ho, 1, c), _BF)
    cols = (jnp.concatenate([zcol, act[:, :, 0:wo - 1]], axis=2),
            act,
            jnp.concatenate([act[:, :, 1:wo], zcol], axis=2))
    for dy in range(3):
        for dx in range(3):
            v = cols[dx]
            if dy == 0:
                v = jnp.concatenate([zrow, v[:, 0:ho - 1]], axis=1)
            elif dy == 2:
                v = jnp.concatenate([v[:, 1:ho], zrow], axis=1)
            blk = (dy * 3 + dx) * c
            imcol_ref[:, :, :, blk:blk + c] = v
    return jnp.dot(imcol_ref[...].reshape(m, 9 * c), wcat_ref[...],
                   preferred_element_type=_F32)


def _net_kernel(x2_ref,
                s10_ref, h10_ref, wc0_ref,
                s20_ref, h20_ref, w20_ref, sw_ref,
                s11_ref, h11_ref, w11_ref, s21_ref, h21_ref, w21_ref,
                s12_ref, h12_ref, w12_ref, s22_ref, h22_ref, w22_ref,
                s13_ref, h13_ref, w13_ref, s23_ref, h23_ref, w23_ref,
                o_ref,
                imA_ref, imB_ref, acts_ref, xbuf_ref):
    b, h, wo, c2 = x2_ref.shape
    ho = h // 2
    cin = c2 // 2
    cout = o_ref.shape[-1]
    m = b * ho * wo

    act = jnp.maximum(x2_ref[...] * s10_ref[...] + h10_ref[...], 0.0)
    act4 = act.reshape(b, ho, 2, wo, c2)
    even = act4[:, :, 0].astype(_BF)
    odd = act4[:, :, 1].astype(_BF)

    acts_ref[...] = even[..., :cin]

    zrow = jnp.zeros((b, 1, wo, c2), _BF)
    zcol = jnp.zeros((b, ho, 1, c2), _BF)
    evenL = jnp.concatenate([zcol, even[:, :, 0:wo - 1]], axis=2)
    oddL = jnp.concatenate([zcol, odd[:, :, 0:wo - 1]], axis=2)
    odd_dn = jnp.concatenate([zrow, odd[:, 0:ho - 1]], axis=1)
    oddL_dn = jnp.concatenate([zrow, oddL[:, 0:ho - 1]], axis=1)
    blocks = (oddL_dn, odd_dn, evenL, even, oddL, odd)
    for kb, v in enumerate(blocks):
        imA_ref[:, :, :, kb * c2:(kb + 1) * c2] = v
    k0 = 6 * c2
    u = jnp.dot(imA_ref[:, :, :, 0:k0].reshape(m, k0), wc0_ref[...],
                preferred_element_type=_F32)

    a2 = jnp.maximum(u.reshape(b, ho, wo, cout) * s20_ref[...] + h20_ref[...],
                     0.0).astype(_BF)
    acc2 = _im2col_dot(a2, imB_ref, w20_ref)
    acc2 = jnp.dot(acts_ref[...].reshape(m, cin), sw_ref[...],
                   preferred_element_type=_F32) + acc2
    xbuf_ref[...] = acc2.reshape(b, ho, wo, cout)

    layers = ((s11_ref, h11_ref, w11_ref, s21_ref, h21_ref, w21_ref),
              (s12_ref, h12_ref, w12_ref, s22_ref, h22_ref, w22_ref),
              (s13_ref, h13_ref, w13_ref, s23_ref, h23_ref, w23_ref))
    for li, (s1, h1, w1, s2, h2, w2) in enumerate(layers):
        p1, p2 = (imA_ref, imB_ref) if li % 2 == 0 else (imB_ref, imA_ref)
        a1 = jnp.maximum(xbuf_ref[...] * s1[...] + h1[...], 0.0).astype(_BF)
        uu = _im2col_dot(a1, p1, w1)
        a2 = jnp.maximum(uu.reshape(b, ho, wo, cout) * s2[...] + h2[...],
                         0.0).astype(_BF)
        vv = _im2col_dot(a2, p2, w2)
        out = vv.reshape(b, ho, wo, cout) + xbuf_ref[...]
        if li == 2:
            o_ref[...] = out.astype(o_ref.dtype)
        else:
            xbuf_ref[...] = out


def _fold_bn(gamma, beta, mean, var):
    scale = gamma / jnp.sqrt(var + _EPS)
    shift = beta - mean * scale
    return scale, shift


def kernel(x, l0_bn1_gamma, l0_bn1_beta, l0_bn1_mean, l0_bn1_var, l0_conv1_w, l0_bn2_gamma, l0_bn2_beta, l0_bn2_mean, l0_bn2_var, l0_conv2_w, l0_short_w, l1_bn1_gamma, l1_bn1_beta, l1_bn1_mean, l1_bn1_var, l1_conv1_w, l1_bn2_gamma, l1_bn2_beta, l1_bn2_mean, l1_bn2_var, l1_conv2_w, l2_bn1_gamma, l2_bn1_beta, l2_bn1_mean, l2_bn1_var, l2_conv1_w, l2_bn2_gamma, l2_bn2_beta, l2_bn2_mean, l2_bn2_var, l2_conv2_w, l3_bn1_gamma, l3_bn1_beta, l3_bn1_mean, l3_bn1_var, l3_conv1_w, l3_bn2_gamma, l3_bn2_beta, l3_bn2_mean, l3_bn2_var, l3_conv2_w):
    n, cin, h, wsp = x.shape
    cout = l0_conv1_w.shape[-1]
    ho, wo = h // 2, wsp // 2
    c2 = 2 * cin
    b = _B

    x2 = jnp.transpose(x, (0, 2, 3, 1)).reshape(n, h, wo, c2)

    s10, h10 = _fold_bn(l0_bn1_gamma, l0_bn1_beta, l0_bn1_mean, l0_bn1_var)
    s10 = jnp.concatenate([s10, s10]).reshape(1, 1, 1, c2)
    h10 = jnp.concatenate([h10, h10]).reshape(1, 1, 1, c2)
    wc0 = jnp.stack(
        [jnp.concatenate([jnp.zeros_like(l0_conv1_w[:, 0]), l0_conv1_w[:, 0]],
                         axis=1),
         jnp.concatenate([l0_conv1_w[:, 1], l0_conv1_w[:, 2]], axis=1)],
        axis=1).reshape(6 * c2, cout).astype(_BF)

    def vec(s):
        return s.reshape(1, 1, 1, -1)

    def wcat(w):
        return w.reshape(-1, w.shape[-1]).astype(_BF)

    s20, h20 = _fold_bn(l0_bn2_gamma, l0_bn2_beta, l0_bn2_mean, l0_bn2_var)
    s11, h11 = _fold_bn(l1_bn1_gamma, l1_bn1_beta, l1_bn1_mean, l1_bn1_var)
    s21, h21 = _fold_bn(l1_bn2_gamma, l1_bn2_beta, l1_bn2_mean, l1_bn2_var)
    s12, h12 = _fold_bn(l2_bn1_gamma, l2_bn1_beta, l2_bn1_mean, l2_bn1_var)
    s22, h22 = _fold_bn(l2_bn2_gamma, l2_bn2_beta, l2_bn2_mean, l2_bn2_var)
    s13, h13 = _fold_bn(l3_bn1_gamma, l3_bn1_beta, l3_bn1_mean, l3_bn1_var)
    s23, h23 = _fold_bn(l3_bn2_gamma, l3_bn2_beta, l3_bn2_mean, l3_bn2_var)

    args = [
        x2,
        s10, h10, wc0,
        vec(s20), vec(h20), wcat(l0_conv2_w), l0_short_w.astype(_BF),
        vec(s11), vec(h11), wcat(l1_conv1_w),
        vec(s21), vec(h21), wcat(l1_conv2_w),
        vec(s12), vec(h12), wcat(l2_conv1_w),
        vec(s22), vec(h22), wcat(l2_conv2_w),
        vec(s13), vec(h13), wcat(l3_conv1_w),
        vec(s23), vec(h23), wcat(l3_conv2_w),
    ]

    def const(shape):
        nd = len(shape)
        return pl.BlockSpec(shape, lambda bi: (0,) * nd)

    in_specs = [pl.BlockSpec((b, h, wo, c2), lambda bi: (bi, 0, 0, 0))]
    in_specs += [const(a.shape) for a in args[1:]]

    out = pl.pallas_call(
        _net_kernel,
        out_shape=jax.ShapeDtypeStruct((n, ho, wo, cout), x.dtype),
        grid=(n // b,),
        in_specs=in_specs,
        out_specs=pl.BlockSpec((b, ho, wo, cout), lambda bi: (bi, 0, 0, 0)),
        scratch_shapes=[
            pltpu.VMEM((b, ho, wo, 9 * cout), _BF),
            pltpu.VMEM((b, ho, wo, 9 * cout), _BF),
            pltpu.VMEM((b, ho, wo, cin), _BF),
            pltpu.VMEM((b, ho, wo, cout), _F32),
        ],
        compiler_params=pltpu.CompilerParams(
            dimension_semantics=("parallel",),
            vmem_limit_bytes=64 * 1024 * 1024,
        ),
    )(*args)

    return jnp.transpose(out, (0, 3, 1, 2))


# speedup vs baseline: 1.5087x; 1.0051x over previous
import jax
import jax.numpy as jnp
from jax.experimental import pallas as pl
from jax.experimental.pallas import tpu as pltpu

_EPS = 1e-5
_B = 4
_STREAMS = 2

_BF = jnp.bfloat16
_F32 = jnp.float32


def _im2col_dot(act, imcol_ref, wcat_ref):
    b, ho, wo, c = act.shape
    m = b * ho * wo
    zrow = jnp.zeros((b, 1, wo, c), _BF)
    zcol = jnp.zeros((b, ho, 1, c), _BF)
    cols = (jnp.concatenate([zcol, act[:, :, 0:wo - 1]], axis=2),
            act,
            jnp.concatenate([act[:, :, 1:wo], zcol], axis=2))
    for dy in range(3):
        for dx in range(3):
            v = cols[dx]
            if dy == 0:
                v = jnp.concatenate([zrow, v[:, 0:ho - 1]], axis=1)
            elif dy == 2:
                v = jnp.concatenate([v[:, 1:ho], zrow], axis=1)
            blk = (dy * 3 + dx) * c
            imcol_ref[:, :, :, blk:blk + c] = v
    return jnp.dot(imcol_ref[...].reshape(m, 9 * c), wcat_ref[...],
                   preferred_element_type=_F32)


def _stream_net(x2v, o_ref, osl, params, imA_ref, imB_ref, acts_ref, xbuf_ref):
    (s10, h10, wc0, s20, h20, w20, sw,
     s11, h11, w11, s21, h21, w21,
     s12, h12, w12, s22, h22, w22,
     s13, h13, w13, s23, h23, w23) = params
    b, h, wo, c2 = x2v.shape
    ho = h // 2
    cin = c2 // 2
    cout = wc0.shape[-1]
    m = b * ho * wo

    act = jnp.maximum(x2v * s10[...] + h10[...], 0.0)
    act4 = act.reshape(b, ho, 2, wo, c2)
    even = act4[:, :, 0].astype(_BF)
    odd = act4[:, :, 1].astype(_BF)

    acts_ref[...] = even[..., :cin]

    zrow = jnp.zeros((b, 1, wo, c2), _BF)
    zcol = jnp.zeros((b, ho, 1, c2), _BF)
    evenL = jnp.concatenate([zcol, even[:, :, 0:wo - 1]], axis=2)
    oddL = jnp.concatenate([zcol, odd[:, :, 0:wo - 1]], axis=2)
    odd_dn = jnp.concatenate([zrow, odd[:, 0:ho - 1]], axis=1)
    oddL_dn = jnp.concatenate([zrow, oddL[:, 0:ho - 1]], axis=1)
    blocks = (oddL_dn, odd_dn, evenL, even, oddL, odd)
    for kb, v in enumerate(blocks):
        imA_ref[:, :, :, kb * c2:(kb + 1) * c2] = v
    k0 = 6 * c2
    u = jnp.dot(imA_ref[:, :, :, 0:k0].reshape(m, k0), wc0[...],
                preferred_element_type=_F32)

    a2 = jnp.maximum(u.reshape(b, ho, wo, cout) * s20[...] + h20[...],
                     0.0).astype(_BF)
    acc2 = _im2col_dot(a2, imB_ref, w20)
    acc2 = jnp.dot(acts_ref[...].reshape(m, cin), sw[...],
                   preferred_element_type=_F32) + acc2
    xbuf_ref[...] = acc2.reshape(b, ho, wo, cout)

    layers = ((s11, h11, w11, s21, h21, w21),
              (s12, h12, w12, s22, h22, w22),
              (s13, h13, w13, s23, h23, w23))
    for li, (s1, h1, w1, s2, h2, w2) in enumerate(layers):
        p1, p2 = (imA_ref, imB_ref) if li % 2 == 0 else (imB_ref, imA_ref)
        a1 = jnp.maximum(xbuf_ref[...] * s1[...] + h1[...], 0.0).astype(_BF)
        uu = _im2col_dot(a1, p1, w1)
        a2 = jnp.maximum(uu.reshape(b, ho, wo, cout) * s2[...] + h2[...],
                         0.0).astype(_BF)
        vv = _im2col_dot(a2, p2, w2)
        out = vv.reshape(b, ho, wo, cout) + xbuf_ref[...]
        if li == 2:
            o_ref[osl] = out.astype(o_ref.dtype)
        else:
            xbuf_ref[...] = out


def _net_kernel(*refs):
    x2_ref = refs[0]
    params = refs[1:26]
    o_ref = refs[26]
    scr = refs[27:]
    b = x2_ref.shape[0]
    hb = b // _STREAMS
    for s in range(_STREAMS):
        sl = slice(s * hb, (s + 1) * hb)
        _stream_net(x2_ref[sl], o_ref, sl, params,
                    scr[4 * s], scr[4 * s + 1], scr[4 * s + 2], scr[4 * s + 3])


def _fold_bn(gamma, beta, mean, var):
    scale = gamma / jnp.sqrt(var + _EPS)
    shift = beta - mean * scale
    return scale, shift


def kernel(x, l0_bn1_gamma, l0_bn1_beta, l0_bn1_mean, l0_bn1_var, l0_conv1_w, l0_bn2_gamma, l0_bn2_beta, l0_bn2_mean, l0_bn2_var, l0_conv2_w, l0_short_w, l1_bn1_gamma, l1_bn1_beta, l1_bn1_mean, l1_bn1_var, l1_conv1_w, l1_bn2_gamma, l1_bn2_beta, l1_bn2_mean, l1_bn2_var, l1_conv2_w, l2_bn1_gamma, l2_bn1_beta, l2_bn1_mean, l2_bn1_var, l2_conv1_w, l2_bn2_gamma, l2_bn2_beta, l2_bn2_mean, l2_bn2_var, l2_conv2_w, l3_bn1_gamma, l3_bn1_beta, l3_bn1_mean, l3_bn1_var, l3_conv1_w, l3_bn2_gamma, l3_bn2_beta, l3_bn2_mean, l3_bn2_var, l3_conv2_w):
    n, cin, h, wsp = x.shape
    cout = l0_conv1_w.shape[-1]
    ho, wo = h // 2, wsp // 2
    c2 = 2 * cin
    b = _B
    hb = b // _STREAMS

    x2 = jnp.transpose(x, (0, 2, 3, 1)).reshape(n, h, wo, c2)

    s10, h10 = _fold_bn(l0_bn1_gamma, l0_bn1_beta, l0_bn1_mean, l0_bn1_var)
    s10 = jnp.concatenate([s10, s10]).reshape(1, 1, 1, c2)
    h10 = jnp.concatenate([h10, h10]).reshape(1, 1, 1, c2)
    wc0 = jnp.stack(
        [jnp.concatenate([jnp.zeros_like(l0_conv1_w[:, 0]), l0_conv1_w[:, 0]],
                         axis=1),
         jnp.concatenate([l0_conv1_w[:, 1], l0_conv1_w[:, 2]], axis=1)],
        axis=1).reshape(6 * c2, cout).astype(_BF)

    def vec(s):
        return s.reshape(1, 1, 1, -1)

    def wcat(w):
        return w.reshape(-1, w.shape[-1]).astype(_BF)

    s20, h20 = _fold_bn(l0_bn2_gamma, l0_bn2_beta, l0_bn2_mean, l0_bn2_var)
    s11, h11 = _fold_bn(l1_bn1_gamma, l1_bn1_beta, l1_bn1_mean, l1_bn1_var)
    s21, h21 = _fold_bn(l1_bn2_gamma, l1_bn2_beta, l1_bn2_mean, l1_bn2_var)
    s12, h12 = _fold_bn(l2_bn1_gamma, l2_bn1_beta, l2_bn1_mean, l2_bn1_var)
    s22, h22 = _fold_bn(l2_bn2_gamma, l2_bn2_beta, l2_bn2_mean, l2_bn2_var)
    s13, h13 = _fold_bn(l3_bn1_gamma, l3_bn1_beta, l3_bn1_mean, l3_bn1_var)
    s23, h23 = _fold_bn(l3_bn2_gamma, l3_bn2_beta, l3_bn2_mean, l3_bn2_var)

    args = [
        x2,
        s10, h10, wc0,
        vec(s20), vec(h20), wcat(l0_conv2_w), l0_short_w.astype(_BF),
        vec(s11), vec(h11), wcat(l1_conv1_w),
        vec(s21), vec(h21), wcat(l1_conv2_w),
        vec(s12), vec(h12), wcat(l2_conv1_w),
        vec(s22), vec(h22), wcat(l2_conv2_w),
        vec(s13), vec(h13), wcat(l3_conv1_w),
        vec(s23), vec(h23), wcat(l3_conv2_w),
    ]

    def const(shape):
        nd = len(shape)
        return pl.BlockSpec(shape, lambda bi: (0,) * nd)

    in_specs = [pl.BlockSpec((b, h, wo, c2), lambda bi: (bi, 0, 0, 0))]
    in_specs += [const(a.shape) for a in args[1:]]

    stream_scratch = [
        pltpu.VMEM((hb, ho, wo, 9 * cout), _BF),
        pltpu.VMEM((hb, ho, wo, 9 * cout), _BF),
        pltpu.VMEM((hb, ho, wo, cin), _BF),
        pltpu.VMEM((hb, ho, wo, cout), _F32),
    ]

    out = pl.pallas_call(
        _net_kernel,
        out_shape=jax.ShapeDtypeStruct((n, ho, wo, cout), x.dtype),
        grid=(n // b,),
        in_specs=in_specs,
        out_specs=pl.BlockSpec((b, ho, wo, cout), lambda bi: (bi, 0, 0, 0)),
        scratch_shapes=stream_scratch * _STREAMS,
        compiler_params=pltpu.CompilerParams(
            dimension_semantics=("parallel",),
            vmem_limit_bytes=64 * 1024 * 1024,
        ),
    )(*args)

    return jnp.transpose(out, (0, 3, 1, 2))
